# Initial kernel scaffold; baseline (speedup 1.0000x reference)
import functools

import jax
import jax.numpy as jnp
from jax import lax
from jax.experimental import pallas as pl

B = 4096
AGENTS = 100
APAD = 104
HID = 256
IG = 5
BBLK = 128


def _body(xs_ref, agp_ref, ws_ref, bs_ref, wu_ref, bu_ref,
          wa1_ref, wa2_ref, wa3_ref, ba_ref, out_ref, ig_ref):
    f32 = jnp.float32
    ag = agp_ref[...]

    locx = ag[:, :, 2:15]
    locy = ag[:, :, 15:28]
    i13 = lax.broadcasted_iota(f32, (BBLK, APAD, 13), 2)
    mx = jnp.max(locx, axis=2, keepdims=True)
    xset = jnp.min(jnp.where(locx == mx, i13, 13.0), axis=2, keepdims=True)
    my = jnp.max(locy, axis=2, keepdims=True)
    yset = jnp.min(jnp.where(locy == my, i13, 13.0), axis=2, keepdims=True)
    dist = jnp.abs(6.0 - xset) + jnp.abs(6.0 - yset)
    aidx = lax.broadcasted_iota(f32, (BBLK, APAD, 1), 1)
    keys = dist * 128.0 + aidx

    rows = []
    for j in range(IG):
        m = jnp.min(keys, axis=1, keepdims=True)
        onehot = keys == m
        row = jnp.sum(jnp.where(onehot, ag, 0.0), axis=1)
        rows.append(row)
        ig_ref[:, j, :] = row
        keys = jnp.where(onehot, keys + 1e9, keys)

    self_info = jnp.maximum(
        jnp.dot(xs_ref[...], ws_ref[...], preferred_element_type=f32)
        + bs_ref[...], 0.0)

    bu = bu_ref[...]
    acc = jnp.zeros((BBLK, HID), f32)
    for t in range(APAD // 8):
        chunk = ag[:, t * 8:(t + 1) * 8, :].reshape(BBLK * 8, 28)
        z = jnp.maximum(
            jnp.dot(chunk, wu_ref[...], preferred_element_type=f32) + bu, 0.0)
        acc = acc + jnp.sum(z.reshape(BBLK, 8, HID), axis=1)
    acc = acc - 4.0 * jnp.maximum(bu, 0.0)

    simp = jnp.zeros((BBLK, HID), f32)
    for j in range(IG):
        simp = simp + jnp.maximum(
            jnp.dot(rows[j], wu_ref[...], preferred_element_type=f32) + bu, 0.0)
    u_sum = acc - simp

    out = jnp.dot(self_info, wa2_ref[...], preferred_element_type=f32)
    out = out + jnp.dot(u_sum, wa3_ref[...], preferred_element_type=f32)
    for j in range(IG):
        out = out + jnp.dot(rows[j], wa1_ref[j], preferred_element_type=f32)
    out_ref[...] = out + ba_ref[...]


@functools.partial(jax.jit, static_argnames=("interpret",))
def _impl(x, Ws, bs, Wu, bu, Wa, ba, interpret=False):
    xs = x[:, :36]
    ag = x[:, 36:].reshape(B, AGENTS, 28)
    agp = jnp.pad(ag, ((0, 0), (0, APAD - AGENTS), (0, 0)))
    wa1 = Wa[: IG * 28].reshape(IG, 28, HID)
    wa2 = Wa[IG * 28: IG * 28 + HID]
    wa3 = Wa[IG * 28 + HID:]
    bs2 = bs.reshape(1, HID)
    bu2 = bu.reshape(1, HID)
    ba2 = ba.reshape(1, HID)

    grid = (B // BBLK,)
    out, ig = pl.pallas_call(
        _body,
        grid=grid,
        in_specs=[
            pl.BlockSpec((BBLK, 36), lambda i: (i, 0)),
            pl.BlockSpec((BBLK, APAD, 28), lambda i: (i, 0, 0)),
            pl.BlockSpec((36, HID), lambda i: (0, 0)),
            pl.BlockSpec((1, HID), lambda i: (0, 0)),
            pl.BlockSpec((28, HID), lambda i: (0, 0)),
            pl.BlockSpec((1, HID), lambda i: (0, 0)),
            pl.BlockSpec((IG, 28, HID), lambda i: (0, 0, 0)),
            pl.BlockSpec((HID, HID), lambda i: (0, 0)),
            pl.BlockSpec((HID, HID), lambda i: (0, 0)),
            pl.BlockSpec((1, HID), lambda i: (0, 0)),
        ],
        out_specs=[
            pl.BlockSpec((BBLK, HID), lambda i: (i, 0)),
            pl.BlockSpec((BBLK, IG, 28), lambda i: (i, 0, 0)),
        ],
        out_shape=[
            jax.ShapeDtypeStruct((B, HID), jnp.float32),
            jax.ShapeDtypeStruct((B, IG, 28), jnp.float32),
        ],
        interpret=interpret,
    )(xs, agp, Ws, bs2, Wu, bu2, wa1, wa2, wa3, ba2)
    return (out, ig)


def kernel(x, Ws, bs, Wu, bu, Wa, ba):
    return _impl(x, Ws, bs, Wu, bu, Wa, ba)


# speedup vs baseline: 11.0574x; 11.0574x over previous
import functools

import jax
import jax.numpy as jnp
from jax import lax
from jax.experimental import pallas as pl

B = 4096
AGENTS = 100
APAD = 104
HID = 256
IG = 5
BBLK = 128


def _body(xs_ref, agp_ref, ws_ref, bs_ref, wu_ref, bu_ref,
          wa1_ref, wa2_ref, wa3_ref, ba_ref, out_ref, ig_ref):
    f32 = jnp.float32
    ag = agp_ref[...]

    locx = ag[:, :, 2:15]
    locy = ag[:, :, 15:28]
    i13 = lax.broadcasted_iota(jnp.int32, (BBLK, APAD, 13), 2).astype(f32)
    mx = jnp.max(locx, axis=2, keepdims=True)
    xset = jnp.min(jnp.where(locx == mx, i13, 13.0), axis=2, keepdims=True)
    my = jnp.max(locy, axis=2, keepdims=True)
    yset = jnp.min(jnp.where(locy == my, i13, 13.0), axis=2, keepdims=True)
    dist = jnp.abs(6.0 - xset) + jnp.abs(6.0 - yset)
    aidx = lax.broadcasted_iota(jnp.int32, (BBLK, APAD, 1), 1).astype(f32)
    keys = dist * 128.0 + aidx

    rows = []
    for j in range(IG):
        m = jnp.min(keys, axis=1, keepdims=True)
        onehot = keys == m
        row = jnp.sum(jnp.where(onehot, ag, 0.0), axis=1)
        rows.append(row)
        ig_ref[:, j, :] = row
        keys = jnp.where(onehot, keys + 1e9, keys)

    self_info = jnp.maximum(
        jnp.dot(xs_ref[...], ws_ref[...], preferred_element_type=f32)
        + bs_ref[...], 0.0)

    bu = bu_ref[...]
    acc = jnp.zeros((BBLK, HID), f32)
    for t in range(APAD // 8):
        chunk = ag[:, t * 8:(t + 1) * 8, :].reshape(BBLK * 8, 28)
        z = jnp.maximum(
            jnp.dot(chunk, wu_ref[...], preferred_element_type=f32) + bu, 0.0)
        acc = acc + jnp.sum(z.reshape(BBLK, 8, HID), axis=1)
    acc = acc - 4.0 * jnp.maximum(bu, 0.0)

    simp = jnp.zeros((BBLK, HID), f32)
    for j in range(IG):
        simp = simp + jnp.maximum(
            jnp.dot(rows[j], wu_ref[...], preferred_element_type=f32) + bu, 0.0)
    u_sum = acc - simp

    out = jnp.dot(self_info, wa2_ref[...], preferred_element_type=f32)
    out = out + jnp.dot(u_sum, wa3_ref[...], preferred_element_type=f32)
    for j in range(IG):
        out = out + jnp.dot(rows[j], wa1_ref[j], preferred_element_type=f32)
    out_ref[...] = out + ba_ref[...]


@functools.partial(jax.jit, static_argnames=("interpret",))
def _impl(x, Ws, bs, Wu, bu, Wa, ba, interpret=False):
    xs = x[:, :36]
    ag = x[:, 36:].reshape(B, AGENTS, 28)
    agp = jnp.pad(ag, ((0, 0), (0, APAD - AGENTS), (0, 0)))
    wa1 = Wa[: IG * 28].reshape(IG, 28, HID)
    wa2 = Wa[IG * 28: IG * 28 + HID]
    wa3 = Wa[IG * 28 + HID:]
    bs2 = bs.reshape(1, HID)
    bu2 = bu.reshape(1, HID)
    ba2 = ba.reshape(1, HID)

    grid = (B // BBLK,)
    out, ig = pl.pallas_call(
        _body,
        grid=grid,
        in_specs=[
            pl.BlockSpec((BBLK, 36), lambda i: (i, 0)),
            pl.BlockSpec((BBLK, APAD, 28), lambda i: (i, 0, 0)),
            pl.BlockSpec((36, HID), lambda i: (0, 0)),
            pl.BlockSpec((1, HID), lambda i: (0, 0)),
            pl.BlockSpec((28, HID), lambda i: (0, 0)),
            pl.BlockSpec((1, HID), lambda i: (0, 0)),
            pl.BlockSpec((IG, 28, HID), lambda i: (0, 0, 0)),
            pl.BlockSpec((HID, HID), lambda i: (0, 0)),
            pl.BlockSpec((HID, HID), lambda i: (0, 0)),
            pl.BlockSpec((1, HID), lambda i: (0, 0)),
        ],
        out_specs=[
            pl.BlockSpec((BBLK, HID), lambda i: (i, 0)),
            pl.BlockSpec((BBLK, IG, 28), lambda i: (i, 0, 0)),
        ],
        out_shape=[
            jax.ShapeDtypeStruct((B, HID), jnp.float32),
            jax.ShapeDtypeStruct((B, IG, 28), jnp.float32),
        ],
        interpret=interpret,
    )(xs, agp, Ws, bs2, Wu, bu2, wa1, wa2, wa3, ba2)
    return (out, ig)


def kernel(x, Ws, bs, Wu, bu, Wa, ba):
    return _impl(x, Ws, bs, Wu, bu, Wa, ba)


# speedup vs baseline: 12.5526x; 1.1352x over previous
import functools

import jax
import jax.numpy as jnp
from jax import lax
from jax.experimental import pallas as pl

B = 4096
AGENTS = 100
APAD = 104
HID = 256
IG = 5
BBLK = 128


def _body(xs_ref, agp_ref, ws_ref, bs_ref, wu_ref, bu_ref,
          wa1_ref, wa2_ref, wa3_ref, ba_ref, out_ref, ig_ref):
    f32 = jnp.float32
    ag = agp_ref[...]

    locx = ag[:, :, 2:15]
    locy = ag[:, :, 15:28]
    i13 = lax.broadcasted_iota(jnp.int32, (BBLK, AGENTS, 13), 2).astype(f32)
    mx = jnp.max(locx, axis=2, keepdims=True)
    xset = jnp.min(jnp.where(locx == mx, i13, 13.0), axis=2, keepdims=True)
    my = jnp.max(locy, axis=2, keepdims=True)
    yset = jnp.min(jnp.where(locy == my, i13, 13.0), axis=2, keepdims=True)
    dist = jnp.abs(6.0 - xset) + jnp.abs(6.0 - yset)
    aidx = lax.broadcasted_iota(jnp.int32, (BBLK, AGENTS, 1), 1).astype(f32)
    keys = dist * 128.0 + aidx

    rows = []
    for j in range(IG):
        m = jnp.min(keys, axis=1, keepdims=True)
        onehot = keys == m
        row = jnp.sum(jnp.where(onehot, ag, 0.0), axis=1)
        rows.append(row)
        ig_ref[:, j, :] = row
        keys = jnp.where(onehot, keys + 1e9, keys)

    self_info = jnp.maximum(
        jnp.dot(xs_ref[...], ws_ref[...], preferred_element_type=f32)
        + bs_ref[...], 0.0)

    bu = bu_ref[...]
    wu = wu_ref[...]
    acc8 = jnp.zeros((BBLK * 8, HID), f32)
    for t in range(AGENTS // 8):
        chunk = ag[:, t * 8:(t + 1) * 8, :].reshape(BBLK * 8, 28)
        acc8 = acc8 + jnp.maximum(
            jnp.dot(chunk, wu, preferred_element_type=f32) + bu, 0.0)
    acc = jnp.sum(acc8.reshape(BBLK, 8, HID), axis=1)
    for k in range(AGENTS % 8):
        acc = acc + jnp.maximum(
            jnp.dot(ag[:, 96 + k, :], wu, preferred_element_type=f32) + bu, 0.0)

    simp = jnp.zeros((BBLK, HID), f32)
    for j in range(IG):
        simp = simp + jnp.maximum(
            jnp.dot(rows[j], wu, preferred_element_type=f32) + bu, 0.0)
    u_sum = acc - simp

    out = jnp.dot(self_info, wa2_ref[...], preferred_element_type=f32)
    out = out + jnp.dot(u_sum, wa3_ref[...], preferred_element_type=f32)
    for j in range(IG):
        out = out + jnp.dot(rows[j], wa1_ref[j], preferred_element_type=f32)
    out_ref[...] = out + ba_ref[...]


@functools.partial(jax.jit, static_argnames=("interpret",))
def _impl(x, Ws, bs, Wu, bu, Wa, ba, interpret=False):
    xs = x[:, :36]
    agp = x[:, 36:].reshape(B, AGENTS, 28)
    wa1 = Wa[: IG * 28].reshape(IG, 28, HID)
    wa2 = Wa[IG * 28: IG * 28 + HID]
    wa3 = Wa[IG * 28 + HID:]
    bs2 = bs.reshape(1, HID)
    bu2 = bu.reshape(1, HID)
    ba2 = ba.reshape(1, HID)

    grid = (B // BBLK,)
    out, ig = pl.pallas_call(
        _body,
        grid=grid,
        in_specs=[
            pl.BlockSpec((BBLK, 36), lambda i: (i, 0)),
            pl.BlockSpec((BBLK, AGENTS, 28), lambda i: (i, 0, 0)),
            pl.BlockSpec((36, HID), lambda i: (0, 0)),
            pl.BlockSpec((1, HID), lambda i: (0, 0)),
            pl.BlockSpec((28, HID), lambda i: (0, 0)),
            pl.BlockSpec((1, HID), lambda i: (0, 0)),
            pl.BlockSpec((IG, 28, HID), lambda i: (0, 0, 0)),
            pl.BlockSpec((HID, HID), lambda i: (0, 0)),
            pl.BlockSpec((HID, HID), lambda i: (0, 0)),
            pl.BlockSpec((1, HID), lambda i: (0, 0)),
        ],
        out_specs=[
            pl.BlockSpec((BBLK, HID), lambda i: (i, 0)),
            pl.BlockSpec((BBLK, IG, 28), lambda i: (i, 0, 0)),
        ],
        out_shape=[
            jax.ShapeDtypeStruct((B, HID), jnp.float32),
            jax.ShapeDtypeStruct((B, IG, 28), jnp.float32),
        ],
        interpret=interpret,
    )(xs, agp, Ws, bs2, Wu, bu2, wa1, wa2, wa3, ba2)
    return (out, ig)


def kernel(x, Ws, bs, Wu, bu, Wa, ba):
    return _impl(x, Ws, bs, Wu, bu, Wa, ba)


# speedup vs baseline: 14.2692x; 1.1368x over previous
import functools

import jax
import jax.numpy as jnp
from jax import lax
from jax.experimental import pallas as pl
from jax.experimental.pallas import tpu as pltpu
from jax.experimental.pallas import tpu_sc as plsc

B = 4096
AGENTS = 100
HID = 256
IG = 5
BBLK = 128
NW = 32
RPW = B // NW
KPAD = 112
BIG = 1e9


def _body_a(xs_ref, ag_ref, ws_ref, bs_ref, wu_ref, bu_ref,
            wa2_ref, ba_ref, keys_ref, part_ref, acc_ref, ag32_ref):
    f32 = jnp.float32
    ag = ag_ref[...]
    locx = ag[:, :, 2:15]
    locy = ag[:, :, 15:28]
    i13 = lax.broadcasted_iota(jnp.int32, (BBLK, AGENTS, 13), 2).astype(f32)
    mx = jnp.max(locx, axis=2, keepdims=True)
    xset = jnp.min(jnp.where(locx == mx, i13, 13.0), axis=2, keepdims=True)
    my = jnp.max(locy, axis=2, keepdims=True)
    yset = jnp.min(jnp.where(locy == my, i13, 13.0), axis=2, keepdims=True)
    dist = jnp.abs(6.0 - xset) + jnp.abs(6.0 - yset)
    aidx = lax.broadcasted_iota(jnp.int32, (BBLK, AGENTS, 1), 1).astype(f32)
    keys2 = (dist * 128.0 + aidx).reshape(BBLK, AGENTS)
    keys_ref[:, :AGENTS] = keys2
    keys_ref[:, AGENTS:] = jnp.full((BBLK, KPAD - AGENTS), BIG, f32)
    ag32_ref[:, :, :28] = ag

    self_info = jnp.maximum(
        jnp.dot(xs_ref[...], ws_ref[...], preferred_element_type=f32)
        + bs_ref[...], 0.0)

    bu = bu_ref[...]
    wu = wu_ref[...]
    acc8 = jnp.zeros((BBLK * 8, HID), f32)
    for t in range(AGENTS // 8):
        chunk = ag[:, t * 8:(t + 1) * 8, :].reshape(BBLK * 8, 28)
        acc8 = acc8 + jnp.maximum(
            jnp.dot(chunk, wu, preferred_element_type=f32) + bu, 0.0)
    acc = jnp.sum(acc8.reshape(BBLK, 8, HID), axis=1)
    for k in range(AGENTS % 8):
        acc = acc + jnp.maximum(
            jnp.dot(ag[:, 96 + k, :], wu, preferred_element_type=f32) + bu, 0.0)

    part_ref[...] = (jnp.dot(self_info, wa2_ref[...], preferred_element_type=f32)
                     + ba_ref[...])
    acc_ref[...] = acc


def _body_b(keys_hbm, agt_hbm, ig_hbm, keys_v, idx_v, rows_v, sem):
    wid = lax.axis_index("s") * 2 + lax.axis_index("c")
    base = wid * RPW

    pltpu.sync_copy(keys_hbm.at[pl.ds(base * KPAD, RPW * KPAD)], keys_v)

    i16 = lax.broadcasted_iota(jnp.int32, (16,), 0)
    perms = [(i16 + d) % 16 for d in (8, 4, 2, 1)]
    bigi = jnp.full((16,), 1 << 20, jnp.int32)
    _dn = lax.GatherDimensionNumbers(
        offset_dims=(), collapsed_slice_dims=(0,), start_index_map=(0,))

    def lane_take(v, p):
        return lax.gather(v, p[:, None], _dn, slice_sizes=(1,),
                          mode=lax.GatherScatterMode.PROMISE_IN_BOUNDS)

    def allmin(v):
        for p in perms:
            v = jnp.minimum(v, lane_take(v, p))
        return v

    def pick5(roff, sel, lane0):
        vs = [keys_v[pl.ds(roff + c * 16, 16)] for c in range(7)]
        for j in range(IG):
            m = vs[0]
            for c in range(1, 7):
                m = jnp.minimum(m, vs[c])
            mj = allmin(m)
            cand = bigi
            for c in range(7):
                cand = jnp.minimum(
                    cand, jnp.where(vs[c] == mj, c * 16 + i16, bigi))
                vs[c] = jnp.where(vs[c] == mj, BIG, vs[c])
            aj = allmin(cand)
            sel = jnp.where(i16 == lane0 + j, aj, sel)
        return sel

    def pair_body(p, _):
        ra = 2 * p
        rb = 2 * p + 1
        sel = jnp.zeros((16,), jnp.int32)
        sel = pick5(ra * KPAD, sel, 0)
        sel = pick5(rb * KPAD, sel, 8)
        gbase = jnp.where(i16 < 8, (base + ra) * AGENTS, (base + rb) * AGENTS)
        tab = gbase + sel
        idx_v[pl.ds(16 * p, 16)] = tab
        return 0

    lax.fori_loop(0, RPW // 2, pair_body, 0, unroll=False)

    pltpu.async_copy(agt_hbm.at[idx_v], rows_v, sem).wait()
    pltpu.sync_copy(rows_v, ig_hbm.at[pl.ds(base * 8, RPW * 8)])


CBLK = 256


def _body_c(part_ref, sall_ref, ig_ref, wu_ref, bu_ref, wa1_ref, wa3_ref,
            out_ref):
    f32 = jnp.float32
    bu = bu_ref[...]
    wu = wu_ref[...]
    out = part_ref[...]
    simp = jnp.zeros((CBLK, HID), f32)
    for j in range(IG):
        row = ig_ref[:, j, :28]
        simp = simp + jnp.maximum(
            jnp.dot(row, wu, preferred_element_type=f32) + bu, 0.0)
        out = out + jnp.dot(row, wa1_ref[j], preferred_element_type=f32)
    u_sum = sall_ref[...] - simp
    out_ref[...] = out + jnp.dot(u_sum, wa3_ref[...], preferred_element_type=f32)


@functools.partial(jax.jit, static_argnames=("interpret",))
def _impl(x, Ws, bs, Wu, bu, Wa, ba, interpret=False):
    f32 = jnp.float32
    xs = x[:, :36]
    ag3 = x[:, 36:].reshape(B, AGENTS, 28)
    wa1 = Wa[: IG * 28].reshape(IG, 28, HID)
    wa2 = Wa[IG * 28: IG * 28 + HID]
    wa3 = Wa[IG * 28 + HID:]
    bs2 = bs.reshape(1, HID)
    bu2 = bu.reshape(1, HID)
    ba2 = ba.reshape(1, HID)

    keys3, part, sall, ag32 = pl.pallas_call(
        _body_a,
        grid=(B // BBLK,),
        in_specs=[
            pl.BlockSpec((BBLK, 36), lambda i: (i, 0)),
            pl.BlockSpec((BBLK, AGENTS, 28), lambda i: (i, 0, 0)),
            pl.BlockSpec((36, HID), lambda i: (0, 0)),
            pl.BlockSpec((1, HID), lambda i: (0, 0)),
            pl.BlockSpec((28, HID), lambda i: (0, 0)),
            pl.BlockSpec((1, HID), lambda i: (0, 0)),
            pl.BlockSpec((HID, HID), lambda i: (0, 0)),
            pl.BlockSpec((1, HID), lambda i: (0, 0)),
        ],
        out_specs=[
            pl.BlockSpec((BBLK, KPAD), lambda i: (i, 0)),
            pl.BlockSpec((BBLK, HID), lambda i: (i, 0)),
            pl.BlockSpec((BBLK, HID), lambda i: (i, 0)),
            pl.BlockSpec((BBLK, AGENTS, 32), lambda i: (i, 0, 0)),
        ],
        out_shape=[
            jax.ShapeDtypeStruct((B, KPAD), f32),
            jax.ShapeDtypeStruct((B, HID), f32),
            jax.ShapeDtypeStruct((B, HID), f32),
            jax.ShapeDtypeStruct((B, AGENTS, 32), f32),
        ],
        interpret=interpret,
    )(xs, ag3, Ws, bs2, Wu, bu2, wa2, ba2)

    keys_flat = keys3.reshape(B * KPAD)

    sc = pl.kernel(
        _body_b,
        mesh=plsc.VectorSubcoreMesh(core_axis_name="c", subcore_axis_name="s"),
        compiler_params=pltpu.CompilerParams(use_tc_tiling_on_sc=False),
        out_type=jax.ShapeDtypeStruct((B * 8, 32), f32),
        scratch_types=[
            pltpu.VMEM((RPW * KPAD,), f32),
            pltpu.VMEM((RPW * 8,), jnp.int32),
            pltpu.VMEM((RPW * 8, 32), f32),
            pltpu.SemaphoreType.DMA,
        ],
    )
    ig8 = sc(keys_flat, ag32.reshape(B * AGENTS, 32)).reshape(B, 8, 32)
    ig3 = ig8[:, :IG, :28]

    out = pl.pallas_call(
        _body_c,
        grid=(B // CBLK,),
        in_specs=[
            pl.BlockSpec((CBLK, HID), lambda i: (i, 0)),
            pl.BlockSpec((CBLK, HID), lambda i: (i, 0)),
            pl.BlockSpec((CBLK, 8, 32), lambda i: (i, 0, 0)),
            pl.BlockSpec((28, HID), lambda i: (0, 0)),
            pl.BlockSpec((1, HID), lambda i: (0, 0)),
            pl.BlockSpec((IG, 28, HID), lambda i: (0, 0, 0)),
            pl.BlockSpec((HID, HID), lambda i: (0, 0)),
        ],
        out_specs=[pl.BlockSpec((CBLK, HID), lambda i: (i, 0))],
        out_shape=[jax.ShapeDtypeStruct((B, HID), f32)],
        interpret=interpret,
    )(part, sall, ig8, Wu, bu2, wa1, wa3)[0]

    return (out, ig3)


def kernel(x, Ws, bs, Wu, bu, Wa, ba):
    return _impl(x, Ws, bs, Wu, bu, Wa, ba)
